# Initial kernel scaffold; baseline (speedup 1.0000x reference)
import jax
import jax.numpy as jnp
from jax.experimental import pallas as pl


def kernel(x, edge_index, W, b):
    raise NotImplementedError("write your pallas kernel here")


# speedup vs baseline: 3.7123x; 3.7123x over previous
import functools

import jax
import jax.numpy as jnp
from jax import lax
from jax.experimental import pallas as pl
from jax.experimental.pallas import tpu as pltpu
from jax.experimental.pallas import tpu_sc as plsc

N = 10000
E = 160000
D = 256
OUT = 256
HALF = OUT // 2
NS = 16
EPT = E // NS
CH = 80
ITERS = EPT // CH
RPT = 624
TAIL = N - NS * RPT


def _matmul(x, W):
    BLK = 1000

    def mm_body(x_ref, w_ref, y0_ref, y1_ref):
        y = jnp.dot(x_ref[...], w_ref[...], preferred_element_type=jnp.float32)
        y0_ref[...] = y[:, :HALF]
        y1_ref[...] = y[:, HALF:]

    return pl.pallas_call(
        mm_body,
        grid=(N // BLK,),
        in_specs=[
            pl.BlockSpec((BLK, D), lambda i: (i, 0)),
            pl.BlockSpec((D, OUT), lambda i: (0, 0)),
        ],
        out_specs=[
            pl.BlockSpec((BLK, HALF), lambda i: (i, 0)),
            pl.BlockSpec((BLK, HALF), lambda i: (i, 0)),
        ],
        out_shape=[
            jax.ShapeDtypeStruct((N, HALF), jnp.float32),
            jax.ShapeDtypeStruct((N, HALF), jnp.float32),
        ],
    )(x, W)


def _make_agg():
    mesh = plsc.VectorSubcoreMesh(core_axis_name="c", subcore_axis_name="s")

    @functools.partial(
        pl.kernel,
        mesh=mesh,
        out_type=[
            jax.ShapeDtypeStruct((N, HALF), jnp.float32),
            jax.ShapeDtypeStruct((N, HALF), jnp.float32),
        ],
        scratch_types=[
            pltpu.VMEM((CH,), jnp.int32),
            pltpu.VMEM((CH,), jnp.int32),
            pltpu.VMEM((CH, HALF), jnp.float32),
            pltpu.VMEM_SHARED((N, HALF), jnp.float32),
            pltpu.SemaphoreType.DMA,
        ],
    )
    def agg(y0_hbm, y1_hbm, src_hbm, dst_hbm, binit0_hbm, binit1_hbm,
            out0_hbm, out1_hbm, srcv, dstv, rows, acc, sem):
        cid = lax.axis_index("c")
        sid = lax.axis_index("s")

        def run(y_hbm, binit_hbm, out_hbm):
            rbase = pl.multiple_of(sid * RPT, 8)
            pltpu.sync_copy(binit_hbm.at[pl.ds(0, RPT)],
                            acc.at[pl.ds(rbase, RPT)])

            @pl.when(sid == NS - 1)
            def _():
                pltpu.sync_copy(binit_hbm.at[pl.ds(0, TAIL)],
                                acc.at[pl.ds(NS * RPT, TAIL)])

            plsc.subcore_barrier()

            def body(j, carry):
                base = sid * EPT + j * CH
                pltpu.sync_copy(src_hbm.at[pl.ds(base, CH)], srcv)
                pltpu.sync_copy(dst_hbm.at[pl.ds(base, CH)], dstv)
                pltpu.async_copy(y_hbm.at[srcv], rows, sem).wait()
                pltpu.sync_copy(rows, acc.at[dstv], add=True)
                return carry

            lax.fori_loop(0, ITERS, body, 0)
            plsc.subcore_barrier()
            pltpu.sync_copy(acc.at[pl.ds(rbase, RPT)],
                            out_hbm.at[pl.ds(rbase, RPT)])

            @pl.when(sid == NS - 1)
            def _():
                pltpu.sync_copy(acc.at[pl.ds(NS * RPT, TAIL)],
                                out_hbm.at[pl.ds(NS * RPT, TAIL)])

        @pl.when(cid == 0)
        def _():
            run(y0_hbm, binit0_hbm, out0_hbm)

        @pl.when(cid == 1)
        def _():
            run(y1_hbm, binit1_hbm, out1_hbm)

    return agg


_agg = _make_agg()


def kernel(x, edge_index, W, b):
    y0, y1 = _matmul(x, W)
    binit0 = jnp.broadcast_to(b[:HALF], (RPT, HALF))
    binit1 = jnp.broadcast_to(b[HALF:], (RPT, HALF))
    out0, out1 = _agg(y0, y1, edge_index[0], edge_index[1], binit0, binit1)
    return jnp.concatenate([out0, out1], axis=1)


# speedup vs baseline: 7.9030x; 2.1289x over previous
import functools

import jax
import jax.numpy as jnp
from jax import lax
from jax.experimental import pallas as pl
from jax.experimental.pallas import tpu as pltpu
from jax.experimental.pallas import tpu_sc as plsc

N = 10000
E = 160000
D = 256
OUT = 256
HALF = OUT // 2
NS = 16
EPT = E // NS
CH = 80
ITERS = EPT // CH
PAIRS = (ITERS - 1) // 2
RPT = 624
TAIL = N - NS * RPT


def _matmul(x, W):
    BLK = 1000

    def mm_body(x_ref, w_ref, y0_ref, y1_ref):
        y = jnp.dot(x_ref[...], w_ref[...], preferred_element_type=jnp.float32)
        y0_ref[...] = y[:, :HALF]
        y1_ref[...] = y[:, HALF:]

    return pl.pallas_call(
        mm_body,
        grid=(N // BLK,),
        in_specs=[
            pl.BlockSpec((BLK, D), lambda i: (i, 0)),
            pl.BlockSpec((D, OUT), lambda i: (0, 0)),
        ],
        out_specs=[
            pl.BlockSpec((BLK, HALF), lambda i: (i, 0)),
            pl.BlockSpec((BLK, HALF), lambda i: (i, 0)),
        ],
        out_shape=[
            jax.ShapeDtypeStruct((N, HALF), jnp.float32),
            jax.ShapeDtypeStruct((N, HALF), jnp.float32),
        ],
    )(x, W)


def _make_agg():
    mesh = plsc.VectorSubcoreMesh(core_axis_name="c", subcore_axis_name="s")

    @functools.partial(
        pl.kernel,
        mesh=mesh,
        out_type=[
            jax.ShapeDtypeStruct((N, HALF), jnp.float32),
            jax.ShapeDtypeStruct((N, HALF), jnp.float32),
        ],
        scratch_types=[
            pltpu.VMEM((ITERS, CH), jnp.int32),
            pltpu.VMEM((CH,), jnp.int32),
            pltpu.VMEM((CH,), jnp.int32),
            pltpu.VMEM((CH, HALF), jnp.float32),
            pltpu.VMEM((CH, HALF), jnp.float32),
            pltpu.VMEM_SHARED((N, HALF), jnp.float32),
            pltpu.SemaphoreType.DMA,
            pltpu.SemaphoreType.DMA,
            pltpu.SemaphoreType.DMA,
            pltpu.SemaphoreType.DMA,
        ],
    )
    def agg(y0_hbm, y1_hbm, src_hbm, dst_hbm, binit0_hbm, binit1_hbm,
            out0_hbm, out1_hbm, srcs, dv_a, dv_b, rows_a, rows_b, acc,
            sem_a, sem_b, sem_da, sem_db):
        cid = lax.axis_index("c")
        sid = lax.axis_index("s")

        def run(y_hbm, binit_hbm, out_hbm):
            pltpu.sync_copy(src_hbm.at[sid], srcs)
            rbase = pl.multiple_of(sid * RPT, 8)
            pltpu.sync_copy(binit_hbm.at[pl.ds(0, RPT)],
                            acc.at[pl.ds(rbase, RPT)])

            @pl.when(sid == NS - 1)
            def _():
                pltpu.sync_copy(binit_hbm.at[pl.ds(0, TAIL)],
                                acc.at[pl.ds(NS * RPT, TAIL)])

            plsc.subcore_barrier()

            def gather(j, buf, sem):
                pltpu.async_copy(y_hbm.at[srcs.at[j]], buf, sem)

            def gwait(j, buf, sem):
                pltpu.make_async_copy(y_hbm.at[srcs.at[j]], buf, sem).wait()

            def dload(j, dbuf, sem):
                base = sid * EPT + j * CH
                pltpu.async_copy(dst_hbm.at[pl.ds(base, CH)], dbuf, sem)

            def dwait(j, dbuf, sem):
                base = sid * EPT + j * CH
                pltpu.make_async_copy(dst_hbm.at[pl.ds(base, CH)], dbuf,
                                      sem).wait()

            def scat(dbuf, buf):
                pltpu.sync_copy(buf, acc.at[dbuf], add=True)

            dload(0, dv_a, sem_da)
            gather(0, rows_a, sem_a)

            def body(g, carry):
                j0 = 2 * g
                dload(j0 + 1, dv_b, sem_db)
                gather(j0 + 1, rows_b, sem_b)
                gwait(j0, rows_a, sem_a)
                dwait(j0, dv_a, sem_da)
                scat(dv_a, rows_a)
                dload(j0 + 2, dv_a, sem_da)
                gather(j0 + 2, rows_a, sem_a)
                gwait(j0 + 1, rows_b, sem_b)
                dwait(j0 + 1, dv_b, sem_db)
                scat(dv_b, rows_b)
                return carry

            lax.fori_loop(0, PAIRS, body, 0)
            gwait(ITERS - 1, rows_a, sem_a)
            dwait(ITERS - 1, dv_a, sem_da)
            scat(dv_a, rows_a)

            plsc.subcore_barrier()
            pltpu.sync_copy(acc.at[pl.ds(rbase, RPT)],
                            out_hbm.at[pl.ds(rbase, RPT)])

            @pl.when(sid == NS - 1)
            def _():
                pltpu.sync_copy(acc.at[pl.ds(NS * RPT, TAIL)],
                                out_hbm.at[pl.ds(NS * RPT, TAIL)])

        @pl.when(cid == 0)
        def _():
            run(y0_hbm, binit0_hbm, out0_hbm)

        @pl.when(cid == 1)
        def _():
            run(y1_hbm, binit1_hbm, out1_hbm)

    return agg


_agg = _make_agg()


def kernel(x, edge_index, W, b):
    y0, y1 = _matmul(x, W)
    src3d = edge_index[0].reshape(NS, ITERS, CH)
    dst1d = edge_index[1]
    binit0 = jnp.broadcast_to(b[:HALF], (RPT, HALF))
    binit1 = jnp.broadcast_to(b[HALF:], (RPT, HALF))
    out0, out1 = _agg(y0, y1, src3d, dst1d, binit0, binit1)
    return jnp.concatenate([out0, out1], axis=1)
